# baseline (speedup 1.0000x reference)
import functools

import jax
import jax.numpy as jnp
from jax import lax
from jax.experimental import pallas as pl
from jax.experimental.pallas import tpu as pltpu
from jax.experimental.pallas import tpu_sc as plsc

VOCAB = 1000000
EMBDIM = 32
B = 4096
L = 200

_NUM_WORKERS = 32
_BB = B // _NUM_WORKERS
_NPART = 4
_HL = L // _NPART
_CH_L = 5
_CHUNK = _CH_L * _BB
_NCH = _HL // _CH_L


@functools.lru_cache(maxsize=None)
def _make_kernel(pitch):

  @functools.partial(
      pl.kernel,
      out_type=jax.ShapeDtypeStruct((L, B, EMBDIM), jnp.float32),
      mesh=plsc.VectorSubcoreMesh(core_axis_name="c", subcore_axis_name="s"),
      compiler_params=pltpu.CompilerParams(
          use_tc_tiling_on_sc=False, needs_layout_passes=False
      ),
      scratch_types=[
          pltpu.VMEM((_BB, _HL, pitch), jnp.int32),
          pltpu.VMEM((_HL * _BB,), jnp.int32),
          pltpu.VMEM((2, _CHUNK, EMBDIM), jnp.float32),
          pltpu.SemaphoreType.DMA((2,)),
          pltpu.SemaphoreType.DMA((2,)),
      ],
  )
  def _emb_gather(idx_hbm, table_hbm, out_hbm, idx_v, idxt_v, rows_v, gsem,
                  osem):
    wid = lax.axis_index("s") * 2 + lax.axis_index("c")
    b0 = wid * _BB
    lanes = lax.iota(jnp.int32, 16)

    def gather_start(offr, b):
      offr = pl.multiple_of(offr, 8)
      pltpu.async_copy(
          table_hbm.at[idxt_v.at[pl.ds(offr, _CHUNK)]],
          rows_v.at[b],
          gsem.at[b],
      )

    def gather_wait(b):
      pltpu.make_async_copy(
          table_hbm.at[idxt_v.at[pl.ds(0, _CHUNK)]],
          rows_v.at[b],
          gsem.at[b],
      ).wait()

    def out_start(l0, b):
      for k in range(_CH_L):
        pltpu.async_copy(
            rows_v.at[b, pl.ds(k * _BB, _BB)],
            out_hbm.at[l0 + jnp.int32(k), pl.ds(b0, _BB)],
            osem.at[b],
        )

    def out_wait(b):
      for k in range(_CH_L):
        pltpu.make_async_copy(
            rows_v.at[b, pl.ds(k * _BB, _BB)],
            out_hbm.at[jnp.int32(k), pl.ds(b0, _BB)],
            osem.at[b],
        ).wait()

    for h in range(_NPART):
      pltpu.sync_copy(
          idx_hbm.at[pl.ds(b0, _BB), pl.ds(h * _HL, _HL)], idx_v
      )

      def tpose(_, carry):
        l, dst = carry
        col = jnp.full((16,), 0, jnp.int32) + l
        for r in range(_BB // 16):
          row_ids = lanes + jnp.int32(r * 16)
          vals = plsc.load_gather(idx_v, [row_ids, col, col - l])
          off = pl.multiple_of(dst + jnp.int32(r * 16), 16)
          idxt_v[pl.ds(off, 16)] = vals
        return (l + jnp.int32(1), dst + jnp.int32(_BB))

      lax.fori_loop(0, _HL, tpose, (jnp.int32(0), jnp.int32(0)),
                    unroll=False)

      gather_start(jnp.int32(0), jnp.int32(0))

      def chunk(_, carry):
        j, offl = carry
        b = j & jnp.int32(1)
        nb = jnp.int32(1) - b

        @pl.when(j >= jnp.int32(1))
        def _():
          out_wait(nb)

        @pl.when(j < jnp.int32(_NCH - 1))
        def _():
          gather_start((offl + jnp.int32(_CH_L)) * jnp.int32(_BB), nb)

        gather_wait(b)
        out_start(jnp.int32(h * _HL) + offl, b)
        return (j + jnp.int32(1), offl + jnp.int32(_CH_L))

      lax.fori_loop(0, _NCH, chunk, (jnp.int32(0), jnp.int32(0)),
                    unroll=False)
      out_wait(jnp.int32((_NCH - 1) & 1))

  return _emb_gather


def kernel(indexseq, table):
  if indexseq.dtype == jnp.int64:
    idx = lax.bitcast_convert_type(indexseq, jnp.int32)
  else:
    idx = jnp.asarray(indexseq, jnp.int32).reshape(B, L, 1)
  return _make_kernel(idx.shape[2])(idx, table)


# speedup vs baseline: 1.7682x; 1.7682x over previous
import functools

import jax
import jax.numpy as jnp
from jax import lax
from jax.experimental import pallas as pl
from jax.experimental.pallas import tpu as pltpu
from jax.experimental.pallas import tpu_sc as plsc

VOCAB = 1000000
EMBDIM = 32
B = 4096
L = 200

_LGROUPS = 8
_BGROUPS = 4
_LL = L // _LGROUPS
_WB = B // _BGROUPS


@functools.partial(
    pl.kernel,
    out_type=jax.ShapeDtypeStruct((L, B, EMBDIM), jnp.float32),
    mesh=plsc.VectorSubcoreMesh(core_axis_name="c", subcore_axis_name="s"),
    compiler_params=pltpu.CompilerParams(use_tc_tiling_on_sc=False),
    scratch_types=[
        pltpu.VMEM((_LL, _WB), jnp.int32),
        pltpu.VMEM((2, _WB, EMBDIM), jnp.float32),
        pltpu.SemaphoreType.DMA((2,)),
        pltpu.SemaphoreType.DMA((2,)),
    ],
)
def _emb_gather(idx_hbm, table_hbm, out_hbm, idx_v, rows_v, gsem, osem):
  wid = lax.axis_index("s") * 2 + lax.axis_index("c")
  lg = wid // jnp.int32(_BGROUPS)
  bg = wid % jnp.int32(_BGROUPS)
  l0 = lg * jnp.int32(_LL)
  b0 = pl.multiple_of(bg * jnp.int32(_WB), _WB)

  pltpu.sync_copy(
      idx_hbm.at[pl.ds(l0, _LL), pl.ds(b0, _WB)], idx_v
  )

  def gather_start(j, b):
    pltpu.async_copy(
        table_hbm.at[idx_v.at[j]],
        rows_v.at[b],
        gsem.at[b],
    )

  def gather_wait(b):
    pltpu.make_async_copy(
        table_hbm.at[idx_v.at[jnp.int32(0)]],
        rows_v.at[b],
        gsem.at[b],
    ).wait()

  def out_start(j, b):
    pltpu.async_copy(
        rows_v.at[b],
        out_hbm.at[l0 + j, pl.ds(b0, _WB)],
        osem.at[b],
    )

  def out_wait(b):
    pltpu.make_async_copy(
        rows_v.at[b],
        out_hbm.at[jnp.int32(0), pl.ds(b0, _WB)],
        osem.at[b],
    ).wait()

  gather_start(jnp.int32(0), jnp.int32(0))

  def chunk(_, j):
    b = j & jnp.int32(1)
    nb = jnp.int32(1) - b

    @pl.when(j >= jnp.int32(1))
    def _():
      out_wait(nb)

    @pl.when(j < jnp.int32(_LL - 1))
    def _():
      gather_start(j + jnp.int32(1), nb)

    gather_wait(b)
    out_start(j, b)
    return j + jnp.int32(1)

  lax.fori_loop(0, _LL, chunk, jnp.int32(0), unroll=False)
  out_wait(jnp.int32((_LL - 1) & 1))


def kernel(indexseq, table):
  idxt = jnp.asarray(indexseq, jnp.int32).T
  return _emb_gather(idxt, table)
